# baseline (speedup 1.0000x reference)
import jax
import jax.numpy as jnp
from jax import lax
from jax.experimental import pallas as pl
from jax.experimental.pallas import tpu as pltpu
from jax.experimental.pallas import tpu_sc as plsc

BATCH = 4096
HIST = 200
WORD_DIM = 32
B = BATCH * HIST
NW = 32
B_CHUNK = 1024
UNITS = (BATCH // B_CHUNK) * HIST
UNITS_PER_W = UNITS // NW
QPH = BATCH // B_CHUNK
TR_STRIDE = B_CHUNK + 1


def _emb_body(idx_hbm, table_hbm, out_hbm, idx_v, rows_v, trt, sem):
    wid = lax.axis_index("s") * 2 + lax.axis_index("c")
    lanes = lax.iota(jnp.int32, 16)

    def step(u, _):
        unit = wid * UNITS_PER_W + u
        h = unit // QPH
        q = unit % QPH
        b0 = q * B_CHUNK
        off = pl.multiple_of(h * BATCH + b0, B_CHUNK)
        pltpu.sync_copy(idx_hbm.at[pl.ds(off, B_CHUNK)], idx_v)
        pltpu.async_copy(table_hbm.at[idx_v], rows_v, sem).wait()

        @plsc.parallel_loop(0, B_CHUNK // 16, unroll=4)
        def _(i16):
            ivec = i16 * 16 + lanes
            btl = i16 // 8
            bbv = (i16 * 16 - btl * 128) + lanes
            for d in range(16):
                for half in range(2):
                    cvec = half * 16 + ((d + lanes) & 15)
                    v = plsc.load_gather(rows_v, [ivec, cvec])
                    plsc.store_scatter(
                        trt,
                        [cvec >> 3, jnp.full((16,), btl, jnp.int32), cvec & 7, bbv],
                        v,
                    )

        for c8 in range(4):
            pltpu.sync_copy(
                trt.at[c8], out_hbm.at[h, c8, pl.ds(q * 8, 8)]
            )
        return 0

    lax.fori_loop(0, UNITS_PER_W, step, 0)


V = 1000000
R_CH = 1024
N_FULL = V // R_CH
N_CH = N_FULL + 1
V_PAD = 1000064
TAIL_R0 = 999040
CH_PER_W = -(-N_CH // NW)
RB_STRIDE = 33


def _tr_body(wem_t_hbm, out_hbm, buf, outbuf, sem):
    wid = lax.axis_index("s") * 2 + lax.axis_index("c")
    lanes = lax.iota(jnp.int32, 16)

    def chunk(k, _):
        cid = wid + k * NW

        @pl.when(cid < N_CH)
        def _():
            r0 = jnp.where(cid == N_FULL, TAIL_R0, cid * R_CH)
            r0 = pl.multiple_of(r0, 128)
            for t in range(32):
                c8, rt = t // 8, t % 8
                pltpu.async_copy(
                    wem_t_hbm.at[
                        pl.ds(c8 * 8, 8), pl.ds(r0 + rt * 128, 128)
                    ],
                    buf.at[t],
                    sem,
                )
            for t in range(32):
                c8, rt = t // 8, t % 8
                pltpu.make_async_copy(
                    wem_t_hbm.at[
                        pl.ds(c8 * 8, 8), pl.ds(r0 + rt * 128, 128)
                    ],
                    buf.at[t],
                    sem,
                ).wait()

            @plsc.parallel_loop(0, R_CH // 16, unroll=4)
            def _(r16):
                rt = r16 // 8
                bb_base = (r16 % 8) * 16
                bbv = bb_base + lanes
                linev = r16 * 4 + (lanes >> 2)
                for d in range(WORD_DIM):
                    cvec = (d + lanes) & 31
                    tv = ((cvec >> 3) << 3) + rt
                    v = plsc.load_gather(buf, [tv, cvec & 7, bbv])
                    colv = ((bbv & 3) << 5) + cvec
                    plsc.store_scatter(outbuf, [linev, colv], v)

            pltpu.sync_copy(
                outbuf, out_hbm.at[pl.ds(pl.multiple_of(r0 // 4, 8), R_CH // 4), :]
            )

        return 0

    lax.fori_loop(0, CH_PER_W, chunk, 0)


@jax.jit
def _table_rm(wem_t):
    return pl.kernel(
        _tr_body,
        out_type=jax.ShapeDtypeStruct((V_PAD * WORD_DIM // 128, 128), jnp.float32),
        mesh=plsc.VectorSubcoreMesh(core_axis_name="c", subcore_axis_name="s"),
        scratch_types=[
            pltpu.VMEM((32, 8, 128), jnp.float32),
            pltpu.VMEM((R_CH // 4, 128), jnp.float32),
            pltpu.SemaphoreType.DMA,
        ],
        compiler_params=pltpu.CompilerParams(
            use_tc_tiling_on_sc=True, needs_layout_passes=False
        ),
    )(wem_t)


@jax.jit
def _emb(idx, word_em):
    return pl.kernel(
        _emb_body,
        out_type=jax.ShapeDtypeStruct(
            (HIST, 4, BATCH // 128, 8, 128), jnp.float32
        ),
        mesh=plsc.VectorSubcoreMesh(core_axis_name="c", subcore_axis_name="s"),
        scratch_types=[
            pltpu.VMEM((B_CHUNK,), jnp.int32),
            pltpu.VMEM((B_CHUNK, WORD_DIM), jnp.float32),
            pltpu.VMEM((4, B_CHUNK // 128, 8, 128), jnp.float32),
            pltpu.SemaphoreType.DMA,
        ],
        compiler_params=pltpu.CompilerParams(
            use_tc_tiling_on_sc=False, needs_layout_passes=False
        ),
    )(idx, word_em)


def kernel(review, word_em):
    idx = review.T.reshape(B).astype(jnp.int32)
    t4 = _table_rm(word_em.T)
    table_rm = t4.reshape(V_PAD, WORD_DIM)
    out6 = _emb(idx, table_rm)
    return out6.transpose(2, 4, 0, 1, 3).reshape(BATCH, HIST, WORD_DIM)


# speedup vs baseline: 1.0509x; 1.0509x over previous
import jax
import jax.numpy as jnp
from jax import lax
from jax.experimental import pallas as pl
from jax.experimental.pallas import tpu as pltpu
from jax.experimental.pallas import tpu_sc as plsc

BATCH = 4096
HIST = 200
WORD_DIM = 32
B = BATCH * HIST
NW = 32
B_CHUNK = 1024
UNITS = (BATCH // B_CHUNK) * HIST
UNITS_PER_W = UNITS // NW
QPH = BATCH // B_CHUNK
TR_STRIDE = B_CHUNK + 1


def _emb_body(idx_hbm, table_hbm, out_hbm, idx_v, rows_v, trt, sem0, sem1):
    wid = lax.axis_index("s") * 2 + lax.axis_index("c")
    lanes = lax.iota(jnp.int32, 16)
    sems = (sem0, sem1)

    def issue(u, slot):
        unit = wid * UNITS_PER_W + u
        off = pl.multiple_of(unit * B_CHUNK, B_CHUNK)
        pltpu.sync_copy(idx_hbm.at[pl.ds(off, B_CHUNK)], idx_v.at[slot])
        pltpu.async_copy(table_hbm.at[idx_v.at[slot]], rows_v.at[slot], sems[slot])

    def finish(u, slot):
        unit = wid * UNITS_PER_W + u
        h = unit // QPH
        q = unit % QPH
        pltpu.make_async_copy(
            table_hbm.at[idx_v.at[slot]], rows_v.at[slot], sems[slot]
        ).wait()

        @plsc.parallel_loop(0, B_CHUNK // 16, unroll=4)
        def _(i16):
            ivec = i16 * 16 + lanes
            btl = i16 // 8
            bbv = (i16 * 16 - btl * 128) + lanes
            for d in range(16):
                for half in range(2):
                    cvec = half * 16 + ((d + lanes) & 15)
                    v = plsc.load_gather(rows_v.at[slot], [ivec, cvec])
                    plsc.store_scatter(
                        trt,
                        [cvec >> 3, jnp.full((16,), btl, jnp.int32), cvec & 7, bbv],
                        v,
                    )

        for c8 in range(4):
            pltpu.sync_copy(
                trt.at[c8], out_hbm.at[h, c8, pl.ds(q * 8, 8)]
            )

    issue(0, 0)

    def pair(g, _):
        issue(2 * g + 1, 1)
        finish(2 * g, 0)
        issue(2 * g + 2, 0)
        finish(2 * g + 1, 1)
        return 0

    lax.fori_loop(0, (UNITS_PER_W - 1) // 2, pair, 0)
    finish(UNITS_PER_W - 1, 0)


V = 1000000
R_CH = 1024
N_FULL = V // R_CH
N_CH = N_FULL + 1
V_PAD = 1000064
TAIL_R0 = 999040
CH_PER_W = -(-N_CH // NW)
RB_STRIDE = 33


def _tr_body(wem_t_hbm, out_hbm, buf, outbuf, sem):
    wid = lax.axis_index("s") * 2 + lax.axis_index("c")
    lanes = lax.iota(jnp.int32, 16)

    def chunk(k, _):
        cid = wid + k * NW

        @pl.when(cid < N_CH)
        def _():
            r0 = jnp.where(cid == N_FULL, TAIL_R0, cid * R_CH)
            r0 = pl.multiple_of(r0, 128)
            for t in range(32):
                c8, rt = t // 8, t % 8
                pltpu.async_copy(
                    wem_t_hbm.at[
                        pl.ds(c8 * 8, 8), pl.ds(r0 + rt * 128, 128)
                    ],
                    buf.at[t],
                    sem,
                )
            for t in range(32):
                c8, rt = t // 8, t % 8
                pltpu.make_async_copy(
                    wem_t_hbm.at[
                        pl.ds(c8 * 8, 8), pl.ds(r0 + rt * 128, 128)
                    ],
                    buf.at[t],
                    sem,
                ).wait()

            @plsc.parallel_loop(0, R_CH // 16, unroll=4)
            def _(r16):
                rt = r16 // 8
                bb_base = (r16 % 8) * 16
                bbv = bb_base + lanes
                linev = r16 * 4 + (lanes >> 2)
                for d in range(WORD_DIM):
                    cvec = (d + lanes) & 31
                    tv = ((cvec >> 3) << 3) + rt
                    v = plsc.load_gather(buf, [tv, cvec & 7, bbv])
                    colv = ((bbv & 3) << 5) + cvec
                    plsc.store_scatter(outbuf, [linev, colv], v)

            pltpu.sync_copy(
                outbuf, out_hbm.at[pl.ds(pl.multiple_of(r0 // 4, 8), R_CH // 4), :]
            )

        return 0

    lax.fori_loop(0, CH_PER_W, chunk, 0)


@jax.jit
def _table_rm(wem_t):
    return pl.kernel(
        _tr_body,
        out_type=jax.ShapeDtypeStruct((V_PAD * WORD_DIM // 128, 128), jnp.float32),
        mesh=plsc.VectorSubcoreMesh(core_axis_name="c", subcore_axis_name="s"),
        scratch_types=[
            pltpu.VMEM((32, 8, 128), jnp.float32),
            pltpu.VMEM((R_CH // 4, 128), jnp.float32),
            pltpu.SemaphoreType.DMA,
        ],
        compiler_params=pltpu.CompilerParams(
            use_tc_tiling_on_sc=True, needs_layout_passes=False
        ),
    )(wem_t)


@jax.jit
def _emb(idx, word_em):
    return pl.kernel(
        _emb_body,
        out_type=jax.ShapeDtypeStruct(
            (HIST, 4, BATCH // 128, 8, 128), jnp.float32
        ),
        mesh=plsc.VectorSubcoreMesh(core_axis_name="c", subcore_axis_name="s"),
        scratch_types=[
            pltpu.VMEM((2, B_CHUNK), jnp.int32),
            pltpu.VMEM((2, B_CHUNK, WORD_DIM), jnp.float32),
            pltpu.VMEM((4, B_CHUNK // 128, 8, 128), jnp.float32),
            pltpu.SemaphoreType.DMA,
            pltpu.SemaphoreType.DMA,
        ],
        compiler_params=pltpu.CompilerParams(
            use_tc_tiling_on_sc=False, needs_layout_passes=False
        ),
    )(idx, word_em)


def kernel(review, word_em):
    idx = review.T.reshape(B).astype(jnp.int32)
    t4 = _table_rm(word_em.T)
    table_rm = t4.reshape(V_PAD, WORD_DIM)
    out6 = _emb(idx, table_rm)
    return out6.transpose(2, 4, 0, 1, 3).reshape(BATCH, HIST, WORD_DIM)


# speedup vs baseline: 1.0656x; 1.0139x over previous
import jax
import jax.numpy as jnp
from jax import lax
from jax.experimental import pallas as pl
from jax.experimental.pallas import tpu as pltpu
from jax.experimental.pallas import tpu_sc as plsc

BATCH = 4096
HIST = 200
WORD_DIM = 32
B = BATCH * HIST
NW = 32
B_CHUNK = 1024
UNITS = (BATCH // B_CHUNK) * HIST
UNITS_PER_W = UNITS // NW
QPH = BATCH // B_CHUNK
TR_STRIDE = B_CHUNK + 1


def _emb_body(idx_hbm, table_hbm, out_hbm, idx_v, rows_v, trt, sem0, sem1):
    wid = lax.axis_index("s") * 2 + lax.axis_index("c")
    lanes = lax.iota(jnp.int32, 16)
    sems = (sem0, sem1)

    def issue(u, slot):
        unit = wid * UNITS_PER_W + u
        off = pl.multiple_of(unit * B_CHUNK, B_CHUNK)
        pltpu.sync_copy(idx_hbm.at[pl.ds(off, B_CHUNK)], idx_v.at[slot])
        pltpu.async_copy(table_hbm.at[idx_v.at[slot]], rows_v.at[slot], sems[slot])

    def finish(u, slot):
        unit = wid * UNITS_PER_W + u
        h = unit // QPH
        q = unit % QPH
        pltpu.make_async_copy(
            table_hbm.at[idx_v.at[slot]], rows_v.at[slot], sems[slot]
        ).wait()

        @plsc.parallel_loop(0, B_CHUNK // 16, unroll=4)
        def _(i16):
            ivec = i16 * 16 + lanes
            btl = i16 // 8
            bbv = (i16 * 16 - btl * 128) + lanes
            for d in range(16):
                for half in range(2):
                    cvec = half * 16 + ((d + lanes) & 15)
                    v = plsc.load_gather(rows_v.at[slot], [ivec, cvec])
                    plsc.store_scatter(
                        trt,
                        [cvec >> 3, jnp.full((16,), btl, jnp.int32), cvec & 7, bbv],
                        v,
                    )

        for c8 in range(4):
            pltpu.sync_copy(
                trt.at[c8], out_hbm.at[h, c8, pl.ds(q * 8, 8)]
            )

    issue(0, 0)

    def pair(g, _):
        issue(2 * g + 1, 1)
        finish(2 * g, 0)
        issue(2 * g + 2, 0)
        finish(2 * g + 1, 1)
        return 0

    lax.fori_loop(0, (UNITS_PER_W - 1) // 2, pair, 0)
    finish(UNITS_PER_W - 1, 0)


V = 1000000
R_CH = 1024
N_FULL = V // R_CH
N_CH = N_FULL + 1
V_PAD = 1000064
TAIL_R0 = 999040
CH_PER_W = -(-N_CH // NW)
RB_STRIDE = 33


def _tr_body(wem_t_hbm, out_hbm, buf, outbuf, sem0, sem1):
    wid = lax.axis_index("s") * 2 + lax.axis_index("c")
    lanes = lax.iota(jnp.int32, 16)

    sems = (sem0, sem1)

    def issue(k, slot):
        cid = wid + k * NW

        @pl.when(cid < N_CH)
        def _():
            r0 = jnp.where(cid == N_FULL, TAIL_R0, cid * R_CH)
            r0 = pl.multiple_of(r0, 128)
            for t in range(32):
                c8, rt = t // 8, t % 8
                pltpu.async_copy(
                    wem_t_hbm.at[
                        pl.ds(c8 * 8, 8), pl.ds(r0 + rt * 128, 128)
                    ],
                    buf.at[slot, t],
                    sems[slot],
                )

    def finish(k, slot):
        cid = wid + k * NW

        @pl.when(cid < N_CH)
        def _():
            r0 = jnp.where(cid == N_FULL, TAIL_R0, cid * R_CH)
            r0 = pl.multiple_of(r0, 128)
            for t in range(32):
                c8, rt = t // 8, t % 8
                pltpu.make_async_copy(
                    wem_t_hbm.at[
                        pl.ds(c8 * 8, 8), pl.ds(r0 + rt * 128, 128)
                    ],
                    buf.at[slot, t],
                    sems[slot],
                ).wait()

            @plsc.parallel_loop(0, R_CH // 16, unroll=4)
            def _(r16):
                rt = r16 // 8
                bb_base = (r16 % 8) * 16
                bbv = bb_base + lanes
                linev = r16 * 4 + (lanes >> 2)
                for d in range(WORD_DIM):
                    cvec = (d + lanes) & 31
                    tv = ((cvec >> 3) << 3) + rt
                    v = plsc.load_gather(buf.at[slot], [tv, cvec & 7, bbv])
                    colv = ((bbv & 3) << 5) + cvec
                    plsc.store_scatter(outbuf, [linev, colv], v)

            pltpu.sync_copy(
                outbuf, out_hbm.at[pl.ds(pl.multiple_of(r0 // 4, 8), R_CH // 4), :]
            )

    issue(0, 0)

    def pair(g, _):
        issue(2 * g + 1, 1)
        finish(2 * g, 0)
        issue(2 * g + 2, 0)
        finish(2 * g + 1, 1)
        return 0

    lax.fori_loop(0, (CH_PER_W - 1) // 2, pair, 0)
    finish(CH_PER_W - 1, 0)


@jax.jit
def _table_rm(wem_t):
    return pl.kernel(
        _tr_body,
        out_type=jax.ShapeDtypeStruct((V_PAD * WORD_DIM // 128, 128), jnp.float32),
        mesh=plsc.VectorSubcoreMesh(core_axis_name="c", subcore_axis_name="s"),
        scratch_types=[
            pltpu.VMEM((2, 32, 8, 128), jnp.float32),
            pltpu.VMEM((R_CH // 4, 128), jnp.float32),
            pltpu.SemaphoreType.DMA,
            pltpu.SemaphoreType.DMA,
        ],
        compiler_params=pltpu.CompilerParams(
            use_tc_tiling_on_sc=True, needs_layout_passes=False
        ),
    )(wem_t)


@jax.jit
def _emb(idx, word_em):
    return pl.kernel(
        _emb_body,
        out_type=jax.ShapeDtypeStruct(
            (HIST, 4, BATCH // 128, 8, 128), jnp.float32
        ),
        mesh=plsc.VectorSubcoreMesh(core_axis_name="c", subcore_axis_name="s"),
        scratch_types=[
            pltpu.VMEM((2, B_CHUNK), jnp.int32),
            pltpu.VMEM((2, B_CHUNK, WORD_DIM), jnp.float32),
            pltpu.VMEM((4, B_CHUNK // 128, 8, 128), jnp.float32),
            pltpu.SemaphoreType.DMA,
            pltpu.SemaphoreType.DMA,
        ],
        compiler_params=pltpu.CompilerParams(
            use_tc_tiling_on_sc=False, needs_layout_passes=False
        ),
    )(idx, word_em)


def kernel(review, word_em):
    idx = review.T.reshape(B).astype(jnp.int32)
    t4 = _table_rm(word_em.T)
    table_rm = t4.reshape(V_PAD, WORD_DIM)
    out6 = _emb(idx, table_rm)
    return out6.transpose(2, 4, 0, 1, 3).reshape(BATCH, HIST, WORD_DIM)
